# baseline (speedup 1.0000x reference)
import functools
import math

import jax
import jax.numpy as jnp
from jax import lax
from jax.experimental import pallas as pl
from jax.experimental.pallas import tpu as pltpu
from jax.experimental.pallas import tpu_sc as plsc

_N, _D, _E, _NEXP = 10000, 128, 320000, 4
_NC, _NS = 2, 16
_NW = _NC * _NS
_EPW = _E // _NW
_K = 80
_NCH = _EPW // _K
_NP = 10240
_RPT = _NP // _NS
_ZR = 128
_DPT = 640
_DEGP = _NS * _DPT

_BN = 1024
_G = (_N + _BN - 1) // _BN

_NOISE_SCALE = 0.3 * math.sqrt(2 * math.log(1.25 / 0.05)) / 1.0

_mesh = plsc.VectorSubcoreMesh(core_axis_name="c", subcore_axis_name="s")


@functools.partial(
    pl.kernel,
    mesh=_mesh,
    out_type=jax.ShapeDtypeStruct((_NC * _DEGP,), jnp.float32),
    scratch_types=[
        pltpu.VMEM((_NCH, _K), jnp.int32),
        pltpu.VMEM((_K,), jnp.float32),
        pltpu.VMEM((_DPT,), jnp.float32),
        pltpu.VMEM_SHARED((_DEGP,), jnp.float32),
    ],
)
def _deg_count(dst3, ones_hbm, zdeg_hbm, out_hbm, didx, ones_v, dbuf, acc):
    c = lax.axis_index("c")
    s = lax.axis_index("s")
    w = c * _NS + s
    pltpu.sync_copy(dst3.at[w], didx)
    pltpu.sync_copy(ones_hbm, ones_v)
    pltpu.sync_copy(zdeg_hbm, dbuf)
    pltpu.sync_copy(dbuf, acc.at[pl.ds(s * _DPT, _DPT)])
    plsc.subcore_barrier()

    def body(i, car):
        pltpu.sync_copy(ones_v, acc.at[didx.at[i]], add=True)
        return car

    lax.fori_loop(0, _NCH, body, 0)
    plsc.subcore_barrier()
    pltpu.sync_copy(acc.at[pl.ds(s * _DPT, _DPT)], dbuf)
    off = pl.multiple_of(c * _DEGP + s * _DPT, 8)
    pltpu.sync_copy(dbuf, out_hbm.at[pl.ds(off, _DPT)])


@functools.partial(
    pl.kernel,
    mesh=_mesh,
    out_type=jax.ShapeDtypeStruct((_NC * _NP, _D), jnp.float32),
    scratch_types=[
        pltpu.VMEM((_NCH, _K), jnp.int32),
        pltpu.VMEM((_K,), jnp.int32),
        pltpu.VMEM((_K,), jnp.int32),
        pltpu.VMEM((_K,), jnp.int32),
        pltpu.VMEM((_K,), jnp.int32),
        pltpu.VMEM((_K, _D), jnp.float32),
        pltpu.VMEM((_K, _D), jnp.float32),
        pltpu.VMEM_SHARED((_NP, _D), jnp.float32),
        pltpu.SemaphoreType.DMA,
        pltpu.SemaphoreType.DMA,
    ],
)
def _agg(sup_hbm, pidx_hbm, zrows_hbm, out_hbm,
         pidx, sidx0, didx0, sidx1, didx1, rows0, rows1, acc, sem0, sem1):
    c = lax.axis_index("c")
    s = lax.axis_index("s")
    w = c * _NS + s
    pltpu.sync_copy(pidx_hbm.at[w], pidx)

    def unpack(i, sbuf, dbuf):
        for t in range(_K // 16):
            p = pidx[i, pl.ds(t * 16, 16)]
            sbuf[pl.ds(t * 16, 16)] = p & 16383
            dbuf[pl.ds(t * 16, 16)] = p >> 14

    unpack(0, sidx0, didx0)
    pltpu.async_copy(sup_hbm.at[sidx0], rows0, sem0)

    pltpu.sync_copy(zrows_hbm, rows1)
    for z in range(_RPT // _K):
        pltpu.sync_copy(rows1, acc.at[pl.ds(s * _RPT + z * _K, _K)])
    plsc.subcore_barrier()

    def body(j, car):
        i0 = 2 * j
        i1 = 2 * j + 1
        unpack(i1, sidx1, didx1)
        pltpu.async_copy(sup_hbm.at[sidx1], rows1, sem1)
        pltpu.make_async_copy(sup_hbm.at[sidx0], rows0, sem0).wait()
        pltpu.sync_copy(rows0, acc.at[didx0], add=True)
        unpack(i0 + 2, sidx0, didx0)
        pltpu.async_copy(sup_hbm.at[sidx0], rows0, sem0)
        pltpu.make_async_copy(sup_hbm.at[sidx1], rows1, sem1).wait()
        pltpu.sync_copy(rows1, acc.at[didx1], add=True)
        return car

    lax.fori_loop(0, _NCH // 2, body, 0)
    pltpu.make_async_copy(sup_hbm.at[sidx0], rows0, sem0).wait()
    pltpu.sync_copy(rows0, acc.at[didx0], add=True)
    plsc.subcore_barrier()
    for z in range(_RPT // _K):
        pltpu.sync_copy(acc.at[pl.ds(s * _RPT + z * _K, _K)], rows0)
        off = pl.multiple_of(c * _NP + s * _RPT + z * _K, 8)
        pltpu.sync_copy(rows0, out_hbm.at[pl.ds(off, _K)])


def _expert_apply(h, eidx, W_ref, b_ref):
    hb = h.astype(jnp.bfloat16)
    hl = (h - hb.astype(jnp.float32)).astype(jnp.bfloat16)
    acc = jnp.zeros_like(h)
    for e in range(_NEXP):
        sel = (eidx == e).astype(jnp.float32)
        We = W_ref[e]
        Wh = We.astype(jnp.bfloat16)
        Wl = (We - Wh.astype(jnp.float32)).astype(jnp.bfloat16)
        d = (jnp.dot(hl, Wh, preferred_element_type=jnp.float32)
             + jnp.dot(hb, Wl, preferred_element_type=jnp.float32)
             + jnp.dot(hb, Wh, preferred_element_type=jnp.float32))
        acc = acc + sel * (d + b_ref[e:e + 1, :])
    return acc


def _prep0_body(x_ref, dinv_ref, eidx_ref, W_ref, b_ref, out_ref):
    h = x_ref[...]
    dinv = dinv_ref[...]
    out_ref[...] = _expert_apply(h, eidx_ref[...], W_ref, b_ref) * dinv


def _prep1_body(aggp_ref, dinv_ref, gdp_ref, eidx_ref, W_ref, b_ref, out_ref):
    agg = aggp_ref[0] + aggp_ref[1]
    dinv = dinv_ref[...]
    h = jnp.maximum(agg * dinv + _NOISE_SCALE * gdp_ref[...], 0.0)
    out_ref[...] = _expert_apply(h, eidx_ref[...], W_ref, b_ref) * dinv


def _prep2_body(aggp_ref, dinv_ref, eidx_ref, W_ref, b_ref, out_ref):
    agg = aggp_ref[0] + aggp_ref[1]
    dinv = dinv_ref[...]
    h = jnp.maximum(agg * dinv, 0.0)
    out_ref[...] = _expert_apply(h, eidx_ref[...], W_ref, b_ref) * dinv


def _final_body(aggp_ref, dinv_ref, out_ref):
    agg = aggp_ref[0] + aggp_ref[1]
    out_ref[...] = agg * dinv_ref[...]


_spec_rows = pl.BlockSpec((_BN, _D), lambda i: (i, 0))
_spec_aggp = pl.BlockSpec((2, _BN, _D), lambda i: (0, i, 0))
_spec_dinv = pl.BlockSpec((_BN, 1), lambda i: (i, 0))
_spec_eidx = pl.BlockSpec((_BN, 1), lambda i: (i, 0))
_spec_W = pl.BlockSpec((_NEXP, _D, _D), lambda i: (0, 0, 0))
_spec_b = pl.BlockSpec((_NEXP, _D), lambda i: (0, 0))
_out_sds = jax.ShapeDtypeStruct((_N, _D), jnp.float32)


def _prep0(x, dinv, eidx, W, b):
    return pl.pallas_call(
        _prep0_body, grid=(_G,),
        in_specs=[_spec_rows, _spec_dinv, _spec_eidx, _spec_W, _spec_b],
        out_specs=_spec_rows, out_shape=_out_sds,
    )(x, dinv, eidx, W, b)


def _prep1(aggp, dinv, gdp, eidx, W, b):
    return pl.pallas_call(
        _prep1_body, grid=(_G,),
        in_specs=[_spec_aggp, _spec_dinv, _spec_rows, _spec_eidx, _spec_W, _spec_b],
        out_specs=_spec_rows, out_shape=_out_sds,
    )(aggp, dinv, gdp, eidx, W, b)


def _prep2(aggp, dinv, eidx, W, b):
    return pl.pallas_call(
        _prep2_body, grid=(_G,),
        in_specs=[_spec_aggp, _spec_dinv, _spec_eidx, _spec_W, _spec_b],
        out_specs=_spec_rows, out_shape=_out_sds,
    )(aggp, dinv, eidx, W, b)


def _final(aggp, dinv):
    return pl.pallas_call(
        _final_body, grid=(_G,),
        in_specs=[_spec_aggp, _spec_dinv],
        out_specs=_spec_rows, out_shape=_out_sds,
    )(aggp, dinv)


def kernel(x, edge_index, W0, b0, wg0, wn0, W1, b1, wg1, wn1, W2, b2, wg2, wn2):
    dst3 = edge_index[1].reshape(_NW, _NCH, _K)
    pidx3 = (edge_index[1] * 16384 + edge_index[0]).reshape(_NW, _NCH, _K)

    kroot = jax.random.key(42)
    k0, k1, k2, kg = jax.random.split(kroot, 4)
    gn0 = jax.random.normal(k0, (_N, _NEXP), dtype=jnp.float32)
    gn1 = jax.random.normal(k1, (_N, _NEXP), dtype=jnp.float32)
    gn2 = jax.random.normal(k2, (_N, _NEXP), dtype=jnp.float32)
    gdp = jax.random.normal(kg, (_N, _D), dtype=jnp.float32)

    ones_e = jnp.ones((_K,), jnp.float32)
    zdeg = jnp.zeros((_DPT,), jnp.float32)
    zrows = jnp.zeros((_K, _D), jnp.float32)

    degp = _deg_count(dst3, ones_e, zdeg).reshape(_NC, _DEGP)
    deg = jnp.maximum(degp[0] + degp[1], 1.0)
    dinv = (deg ** -0.5).reshape(_DEGP, 1)

    def gate_idx(h, wg, wn, gn):
        clean = h @ wg
        noise_std = jax.nn.softplus(h @ wn) + 1e-2
        noisy = clean + gn * noise_std
        _, ti = lax.top_k(noisy, 1)
        return ti

    dinv_n = dinv[:_N]

    sup0 = _prep0(x, dinv, gate_idx(x, wg0, wn0, gn0), W0, b0)
    aggp0 = _agg(sup0, pidx3, zrows).reshape(_NC, _NP, _D)
    h1 = jnp.maximum((aggp0[0, :_N] + aggp0[1, :_N]) * dinv_n
                     + _NOISE_SCALE * gdp, 0.0)
    sup1 = _prep1(aggp0, dinv, gdp, gate_idx(h1, wg1, wn1, gn1), W1, b1)
    aggp1 = _agg(sup1, pidx3, zrows).reshape(_NC, _NP, _D)
    h2 = jnp.maximum((aggp1[0, :_N] + aggp1[1, :_N]) * dinv_n, 0.0)
    sup2 = _prep2(aggp1, dinv, gate_idx(h2, wg2, wn2, gn2), W2, b2)
    aggp2 = _agg(sup2, pidx3, zrows).reshape(_NC, _NP, _D)
    return _final(aggp2, dinv)


# speedup vs baseline: 5.0984x; 5.0984x over previous
import functools
import math

import jax
import jax.numpy as jnp
from jax import lax
from jax.experimental import pallas as pl
from jax.experimental.pallas import tpu as pltpu
from jax.experimental.pallas import tpu_sc as plsc

_N, _D, _E, _NEXP = 10000, 128, 320000, 4
_NC, _NS = 2, 16
_NW = _NC * _NS
_EPW = _E // _NW
_K = 80
_NCH = _EPW // _K
_NP = 10240
_RPT = _NP // _NS
_ZR = 128
_DPT = 640
_DEGP = _NS * _DPT

_BN = 1024
_G = (_N + _BN - 1) // _BN

_NOISE_SCALE = 0.3 * math.sqrt(2 * math.log(1.25 / 0.05)) / 1.0

_mesh = plsc.VectorSubcoreMesh(core_axis_name="c", subcore_axis_name="s")


@functools.partial(
    pl.kernel,
    mesh=_mesh,
    out_type=jax.ShapeDtypeStruct((_NC * _DEGP,), jnp.float32),
    scratch_types=[
        pltpu.VMEM((_NCH, _K), jnp.int32),
        pltpu.VMEM((_K,), jnp.float32),
        pltpu.VMEM((_DPT,), jnp.float32),
        pltpu.VMEM_SHARED((_DEGP,), jnp.float32),
    ],
)
def _deg_count(dst3, ones_hbm, zdeg_hbm, out_hbm, didx, ones_v, dbuf, acc):
    c = lax.axis_index("c")
    s = lax.axis_index("s")
    w = c * _NS + s
    pltpu.sync_copy(dst3.at[w], didx)
    pltpu.sync_copy(ones_hbm, ones_v)
    pltpu.sync_copy(zdeg_hbm, dbuf)
    pltpu.sync_copy(dbuf, acc.at[pl.ds(s * _DPT, _DPT)])
    plsc.subcore_barrier()

    def body(i, car):
        pltpu.sync_copy(ones_v, acc.at[didx.at[i]], add=True)
        return car

    lax.fori_loop(0, _NCH, body, 0)
    plsc.subcore_barrier()
    pltpu.sync_copy(acc.at[pl.ds(s * _DPT, _DPT)], dbuf)
    off = pl.multiple_of(c * _DEGP + s * _DPT, 8)
    pltpu.sync_copy(dbuf, out_hbm.at[pl.ds(off, _DPT)])


@functools.partial(
    pl.kernel,
    mesh=_mesh,
    out_type=jax.ShapeDtypeStruct((_NC * _NP, _D), jnp.float32),
    scratch_types=[
        pltpu.VMEM((_NCH, _K), jnp.int32),
        pltpu.VMEM((_K,), jnp.int32),
        pltpu.VMEM((_K,), jnp.int32),
        pltpu.VMEM((_K,), jnp.int32),
        pltpu.VMEM((_K,), jnp.int32),
        pltpu.VMEM((_K, _D), jnp.float32),
        pltpu.VMEM((_K, _D), jnp.float32),
        pltpu.VMEM_SHARED((_NP, _D), jnp.float32),
        pltpu.SemaphoreType.DMA,
        pltpu.SemaphoreType.DMA,
    ],
)
def _agg(sup_hbm, pidx_hbm, zrows_hbm, out_hbm,
         pidx, sidx0, didx0, sidx1, didx1, rows0, rows1, acc, sem0, sem1):
    c = lax.axis_index("c")
    s = lax.axis_index("s")
    w = c * _NS + s
    pltpu.sync_copy(pidx_hbm.at[w], pidx)

    def unpack(i, sbuf, dbuf):
        for t in range(_K // 16):
            p = pidx[i, pl.ds(t * 16, 16)]
            sbuf[pl.ds(t * 16, 16)] = p & 16383
            dbuf[pl.ds(t * 16, 16)] = p >> 14

    unpack(0, sidx0, didx0)
    pltpu.async_copy(sup_hbm.at[sidx0], rows0, sem0)

    pltpu.sync_copy(zrows_hbm, rows1)
    for z in range(_RPT // _K):
        pltpu.sync_copy(rows1, acc.at[pl.ds(s * _RPT + z * _K, _K)])
    plsc.subcore_barrier()

    def body(j, car):
        i0 = 2 * j
        i1 = 2 * j + 1
        unpack(i1, sidx1, didx1)
        pltpu.async_copy(sup_hbm.at[sidx1], rows1, sem1)
        pltpu.make_async_copy(sup_hbm.at[sidx0], rows0, sem0).wait()
        pltpu.sync_copy(rows0, acc.at[didx0], add=True)
        unpack(i0 + 2, sidx0, didx0)
        pltpu.async_copy(sup_hbm.at[sidx0], rows0, sem0)
        pltpu.make_async_copy(sup_hbm.at[sidx1], rows1, sem1).wait()
        pltpu.sync_copy(rows1, acc.at[didx1], add=True)
        return car

    lax.fori_loop(0, _NCH // 2, body, 0)
    pltpu.make_async_copy(sup_hbm.at[sidx0], rows0, sem0).wait()
    pltpu.sync_copy(rows0, acc.at[didx0], add=True)
    plsc.subcore_barrier()
    for z in range(_RPT // _K):
        pltpu.sync_copy(acc.at[pl.ds(s * _RPT + z * _K, _K)], rows0)
        off = pl.multiple_of(c * _NP + s * _RPT + z * _K, 8)
        pltpu.sync_copy(rows0, out_hbm.at[pl.ds(off, _K)])


def _gate_and_expert(h, gn, wg_ref, wn_ref, W_ref, b_ref):
    hb = h.astype(jnp.bfloat16)
    clean = jnp.dot(hb, wg_ref[...].astype(jnp.bfloat16),
                    preferred_element_type=jnp.float32)
    stdl = jnp.dot(hb, wn_ref[...].astype(jnp.bfloat16),
                   preferred_element_type=jnp.float32)
    std = jax.nn.softplus(stdl) + 1e-2
    noisy = clean + gn * std
    best = noisy[:, 0:1]
    eidx = jnp.zeros_like(best, dtype=jnp.int32)
    for j in range(1, _NEXP):
        vj = noisy[:, j:j + 1]
        m = vj > best
        best = jnp.where(m, vj, best)
        eidx = jnp.where(m, j, eidx)
    return _expert_apply(h, eidx, W_ref, b_ref)


def _expert_apply(h, eidx, W_ref, b_ref):
    hb = h.astype(jnp.bfloat16)
    hl = (h - hb.astype(jnp.float32)).astype(jnp.bfloat16)
    acc = jnp.zeros_like(h)
    for e in range(_NEXP):
        sel = (eidx == e).astype(jnp.float32)
        We = W_ref[e]
        Wh = We.astype(jnp.bfloat16)
        Wl = (We - Wh.astype(jnp.float32)).astype(jnp.bfloat16)
        d = (jnp.dot(hl, Wh, preferred_element_type=jnp.float32)
             + jnp.dot(hb, Wl, preferred_element_type=jnp.float32)
             + jnp.dot(hb, Wh, preferred_element_type=jnp.float32))
        acc = acc + sel * (d + b_ref[e:e + 1, :])
    return acc


def _prep0_body(x_ref, dinv_ref, gn_ref, wg_ref, wn_ref, W_ref, b_ref, out_ref):
    h = x_ref[...]
    dinv = dinv_ref[...]
    out_ref[...] = _gate_and_expert(h, gn_ref[...], wg_ref, wn_ref, W_ref, b_ref) * dinv


def _prep1_body(aggp_ref, dinv_ref, gdp_ref, gn_ref, wg_ref, wn_ref, W_ref, b_ref, out_ref):
    agg = aggp_ref[0] + aggp_ref[1]
    dinv = dinv_ref[...]
    h = jnp.maximum(agg * dinv + _NOISE_SCALE * gdp_ref[...], 0.0)
    out_ref[...] = _gate_and_expert(h, gn_ref[...], wg_ref, wn_ref, W_ref, b_ref) * dinv


def _prep2_body(aggp_ref, dinv_ref, gn_ref, wg_ref, wn_ref, W_ref, b_ref, out_ref):
    agg = aggp_ref[0] + aggp_ref[1]
    dinv = dinv_ref[...]
    h = jnp.maximum(agg * dinv, 0.0)
    out_ref[...] = _gate_and_expert(h, gn_ref[...], wg_ref, wn_ref, W_ref, b_ref) * dinv


def _final_body(aggp_ref, dinv_ref, out_ref):
    agg = aggp_ref[0] + aggp_ref[1]
    out_ref[...] = agg * dinv_ref[...]


_spec_rows = pl.BlockSpec((_BN, _D), lambda i: (i, 0))
_spec_aggp = pl.BlockSpec((2, _BN, _D), lambda i: (0, i, 0))
_spec_dinv = pl.BlockSpec((_BN, 1), lambda i: (i, 0))
_spec_gn = pl.BlockSpec((_BN, _NEXP), lambda i: (i, 0))
_spec_wg = pl.BlockSpec((_D, _NEXP), lambda i: (0, 0))
_spec_W = pl.BlockSpec((_NEXP, _D, _D), lambda i: (0, 0, 0))
_spec_b = pl.BlockSpec((_NEXP, _D), lambda i: (0, 0))
_out_sds = jax.ShapeDtypeStruct((_N, _D), jnp.float32)


def _prep0(x, dinv, gn, wg, wn, W, b):
    return pl.pallas_call(
        _prep0_body, grid=(_G,),
        in_specs=[_spec_rows, _spec_dinv, _spec_gn, _spec_wg, _spec_wg, _spec_W, _spec_b],
        out_specs=_spec_rows, out_shape=_out_sds,
    )(x, dinv, gn, wg, wn, W, b)


def _prep1(aggp, dinv, gdp, gn, wg, wn, W, b):
    return pl.pallas_call(
        _prep1_body, grid=(_G,),
        in_specs=[_spec_aggp, _spec_dinv, _spec_rows, _spec_gn, _spec_wg,
                  _spec_wg, _spec_W, _spec_b],
        out_specs=_spec_rows, out_shape=_out_sds,
    )(aggp, dinv, gdp, gn, wg, wn, W, b)


def _prep2(aggp, dinv, gn, wg, wn, W, b):
    return pl.pallas_call(
        _prep2_body, grid=(_G,),
        in_specs=[_spec_aggp, _spec_dinv, _spec_gn, _spec_wg, _spec_wg, _spec_W, _spec_b],
        out_specs=_spec_rows, out_shape=_out_sds,
    )(aggp, dinv, gn, wg, wn, W, b)


def _final(aggp, dinv):
    return pl.pallas_call(
        _final_body, grid=(_G,),
        in_specs=[_spec_aggp, _spec_dinv],
        out_specs=_spec_rows, out_shape=_out_sds,
    )(aggp, dinv)


def kernel(x, edge_index, W0, b0, wg0, wn0, W1, b1, wg1, wn1, W2, b2, wg2, wn2):
    dst3 = edge_index[1].reshape(_NW, _NCH, _K)
    pidx3 = (edge_index[1] * 16384 + edge_index[0]).reshape(_NW, _NCH, _K)

    kroot = jax.random.key(42)
    k0, k1, k2, kg = jax.random.split(kroot, 4)
    gn0 = jax.random.normal(k0, (_N, _NEXP), dtype=jnp.float32)
    gn1 = jax.random.normal(k1, (_N, _NEXP), dtype=jnp.float32)
    gn2 = jax.random.normal(k2, (_N, _NEXP), dtype=jnp.float32)
    gdp = jax.random.normal(kg, (_N, _D), dtype=jnp.float32)

    ones_e = jnp.ones((_K,), jnp.float32)
    zdeg = jnp.zeros((_DPT,), jnp.float32)
    zrows = jnp.zeros((_K, _D), jnp.float32)

    degp = _deg_count(dst3, ones_e, zdeg).reshape(_NC, _DEGP)
    deg = jnp.maximum(degp[0] + degp[1], 1.0)
    dinv = (deg ** -0.5).reshape(_DEGP, 1)

    sup0 = _prep0(x, dinv, gn0, wg0, wn0, W0, b0)
    aggp0 = _agg(sup0, pidx3, zrows).reshape(_NC, _NP, _D)
    sup1 = _prep1(aggp0, dinv, gdp, gn1, wg1, wn1, W1, b1)
    aggp1 = _agg(sup1, pidx3, zrows).reshape(_NC, _NP, _D)
    sup2 = _prep2(aggp1, dinv, gn2, wg2, wn2, W2, b2)
    aggp2 = _agg(sup2, pidx3, zrows).reshape(_NC, _NP, _D)
    return _final(aggp2, dinv)
